# Initial kernel scaffold; baseline (speedup 1.0000x reference)
import jax
import jax.numpy as jnp
from jax.experimental import pallas as pl


def kernel(hidden_states, gate_weight, e_score_correction_bias, expert_gate_w, expert_up_w, expert_down_w, shared_gate_w, shared_up_w, shared_down_w):
    raise NotImplementedError("write your pallas kernel here")


# speedup vs baseline: 1.1581x; 1.1581x over previous
import functools

import jax
import jax.numpy as jnp
from jax.experimental import pallas as pl
from jax.experimental.pallas import tpu as pltpu

E = 8
TOPK = 2
NG = 2
GS = E // NG
D = 1024
FF = 512
RSF = 2.5

TB = 256


def _dot_t(a, b):
    return jax.lax.dot_general(a, b, (((1,), (1,)), ((), ())),
                               preferred_element_type=jnp.float32)


def _moe_body(x_ref, gw_ref, b_ref, eg_ref, eu_ref, ed_ref,
              sg_ref, su_ref, sd_ref, o_ref, comb_ref):
    e = pl.program_id(1)
    cols = jax.lax.broadcasted_iota(jnp.int32, (TB, E), 1)

    @pl.when(e == 0)
    def _init():
        x = x_ref[...]
        logits = _dot_t(x, gw_ref[...])
        scores = jax.nn.sigmoid(logits)
        s_corr = scores + b_ref[...]
        neg = jnp.float32(-jnp.inf)

        def top2sum(g):
            mask = (cols // GS) == g
            vals = jnp.where(mask, s_corr, neg)
            m1 = jnp.max(vals, axis=1, keepdims=True)
            eq = vals == m1
            neq = jnp.sum(eq.astype(jnp.float32), axis=1, keepdims=True)
            m2 = jnp.where(neq >= 2.0, m1,
                           jnp.max(jnp.where(eq, neg, vals), axis=1,
                                   keepdims=True))
            return m1 + m2

        gs0 = top2sum(0)
        gs1 = top2sum(1)
        sel = jnp.where(gs0 >= gs1, 0, 1)
        smask = (cols // GS) == sel
        v = jnp.where(smask, s_corr, 0.0)

        big = jnp.int32(E + 1)
        m1v = jnp.max(v, axis=1, keepdims=True)
        idx1 = jnp.min(jnp.where(v == m1v, cols, big), axis=1, keepdims=True)
        is1 = cols == idx1
        w1 = jnp.sum(jnp.where(is1, scores, 0.0), axis=1, keepdims=True)
        v2 = jnp.where(is1, neg, v)
        m2v = jnp.max(v2, axis=1, keepdims=True)
        idx2 = jnp.min(jnp.where(v2 == m2v, cols, big), axis=1, keepdims=True)
        is2 = cols == idx2
        w2 = jnp.sum(jnp.where(is2, scores, 0.0), axis=1, keepdims=True)

        scale = jnp.float32(RSF) / (w1 + w2 + 1e-20)
        comb_ref[...] = (jnp.where(is1, w1, 0.0)
                         + jnp.where(is2, w2, 0.0)) * scale

        hg = _dot_t(x, sg_ref[...])
        hu = _dot_t(x, su_ref[...])
        o_ref[...] = _dot_t(jax.nn.silu(hg) * hu, sd_ref[...])

    x = x_ref[...]
    hg = _dot_t(x, eg_ref[0])
    hu = _dot_t(x, eu_ref[0])
    h = jax.nn.silu(hg) * hu
    eo = _dot_t(h, ed_ref[0])
    ce = jnp.sum(jnp.where(cols == e, comb_ref[...], 0.0), axis=1,
                 keepdims=True)
    o_ref[...] = o_ref[...] + eo * ce


@functools.partial(jax.jit, static_argnames=("interpret",))
def _moe(x, gate_weight, bias2d, eg, eu, ed, sg, su, sd, interpret=False):
    T = x.shape[0]
    nblk = T // TB
    grid = (nblk, E)
    out = pl.pallas_call(
        _moe_body,
        grid=grid,
        in_specs=[
            pl.BlockSpec((TB, D), lambda i, e: (i, 0)),
            pl.BlockSpec((E, D), lambda i, e: (0, 0)),
            pl.BlockSpec((1, E), lambda i, e: (0, 0)),
            pl.BlockSpec((1, FF, D), lambda i, e: (e, 0, 0)),
            pl.BlockSpec((1, FF, D), lambda i, e: (e, 0, 0)),
            pl.BlockSpec((1, D, FF), lambda i, e: (e, 0, 0)),
            pl.BlockSpec((FF, D), lambda i, e: (0, 0)),
            pl.BlockSpec((FF, D), lambda i, e: (0, 0)),
            pl.BlockSpec((D, FF), lambda i, e: (0, 0)),
        ],
        out_specs=pl.BlockSpec((TB, D), lambda i, e: (i, 0)),
        out_shape=jax.ShapeDtypeStruct((T, D), jnp.float32),
        scratch_shapes=[pltpu.VMEM((TB, E), jnp.float32)],
        compiler_params=pltpu.CompilerParams(
            dimension_semantics=("parallel", "arbitrary"),
        ),
        interpret=interpret,
    )(x, gate_weight, bias2d, eg, eu, ed, sg, su, sd)
    return out


def kernel(hidden_states, gate_weight, e_score_correction_bias,
           expert_gate_w, expert_up_w, expert_down_w,
           shared_gate_w, shared_up_w, shared_down_w):
    orig_shape = hidden_states.shape
    x = hidden_states.reshape(-1, D).astype(jnp.float32)
    out = _moe(x, gate_weight, e_score_correction_bias.reshape(1, E),
               expert_gate_w, expert_up_w, expert_down_w,
               shared_gate_w, shared_up_w, shared_down_w)
    return out.reshape(orig_shape)


# speedup vs baseline: 1.1621x; 1.0034x over previous
import functools

import jax
import jax.numpy as jnp
from jax.experimental import pallas as pl
from jax.experimental.pallas import tpu as pltpu

E = 8
TOPK = 2
NG = 2
GS = E // NG
D = 1024
FF = 512
RSF = 2.5

TB = 256


def _dot_t(a, b):
    return jax.lax.dot_general(a, b, (((1,), (1,)), ((), ())),
                               preferred_element_type=jnp.float32)


def _moe_body(x_ref, gw_ref, b_ref, eg_ref, eu_ref, ed_ref,
              sg_ref, su_ref, sd_ref, o_ref, comb_ref):
    e = pl.program_id(1)
    cols = jax.lax.broadcasted_iota(jnp.int32, (TB, E), 1)

    @pl.when(e == 0)
    def _init():
        x = x_ref[...]
        logits = _dot_t(x, gw_ref[...])
        scores = jax.nn.sigmoid(logits)
        s_corr = scores + b_ref[...]
        neg = jnp.float32(-jnp.inf)

        def top2sum(g):
            mask = (cols // GS) == g
            vals = jnp.where(mask, s_corr, neg)
            m1 = jnp.max(vals, axis=1, keepdims=True)
            eq = vals == m1
            neq = jnp.sum(eq.astype(jnp.float32), axis=1, keepdims=True)
            m2 = jnp.where(neq >= 2.0, m1,
                           jnp.max(jnp.where(eq, neg, vals), axis=1,
                                   keepdims=True))
            return m1 + m2

        gs0 = top2sum(0)
        gs1 = top2sum(1)
        sel = jnp.where(gs0 >= gs1, 0, 1)
        smask = (cols // GS) == sel
        v = jnp.where(smask, s_corr, 0.0)

        big = jnp.int32(E + 1)
        m1v = jnp.max(v, axis=1, keepdims=True)
        idx1 = jnp.min(jnp.where(v == m1v, cols, big), axis=1, keepdims=True)
        is1 = cols == idx1
        w1 = jnp.sum(jnp.where(is1, scores, 0.0), axis=1, keepdims=True)
        v2 = jnp.where(is1, neg, v)
        m2v = jnp.max(v2, axis=1, keepdims=True)
        idx2 = jnp.min(jnp.where(v2 == m2v, cols, big), axis=1, keepdims=True)
        is2 = cols == idx2
        w2 = jnp.sum(jnp.where(is2, scores, 0.0), axis=1, keepdims=True)

        scale = jnp.float32(RSF) / (w1 + w2 + 1e-20)
        comb_ref[...] = (jnp.where(is1, w1, 0.0)
                         + jnp.where(is2, w2, 0.0)) * scale

        hg = _dot_t(x, sg_ref[...])
        hu = _dot_t(x, su_ref[...])
        o_ref[...] = _dot_t(jax.nn.silu(hg) * hu, sd_ref[...])

    x = x_ref[...].astype(jnp.bfloat16)
    hg = _dot_t(x, eg_ref[0].astype(jnp.bfloat16))
    hu = _dot_t(x, eu_ref[0].astype(jnp.bfloat16))
    h = jax.nn.silu(hg) * hu
    eo = _dot_t(h.astype(jnp.bfloat16), ed_ref[0].astype(jnp.bfloat16))
    ce = jnp.sum(jnp.where(cols == e, comb_ref[...], 0.0), axis=1,
                 keepdims=True)
    o_ref[...] = o_ref[...] + eo * ce


@functools.partial(jax.jit, static_argnames=("interpret",))
def _moe(x, gate_weight, bias2d, eg, eu, ed, sg, su, sd, interpret=False):
    T = x.shape[0]
    nblk = T // TB
    grid = (nblk, E)
    out = pl.pallas_call(
        _moe_body,
        grid=grid,
        in_specs=[
            pl.BlockSpec((TB, D), lambda i, e: (i, 0)),
            pl.BlockSpec((E, D), lambda i, e: (0, 0)),
            pl.BlockSpec((1, E), lambda i, e: (0, 0)),
            pl.BlockSpec((1, FF, D), lambda i, e: (e, 0, 0)),
            pl.BlockSpec((1, FF, D), lambda i, e: (e, 0, 0)),
            pl.BlockSpec((1, D, FF), lambda i, e: (e, 0, 0)),
            pl.BlockSpec((FF, D), lambda i, e: (0, 0)),
            pl.BlockSpec((FF, D), lambda i, e: (0, 0)),
            pl.BlockSpec((D, FF), lambda i, e: (0, 0)),
        ],
        out_specs=pl.BlockSpec((TB, D), lambda i, e: (i, 0)),
        out_shape=jax.ShapeDtypeStruct((T, D), jnp.float32),
        scratch_shapes=[pltpu.VMEM((TB, E), jnp.float32)],
        compiler_params=pltpu.CompilerParams(
            dimension_semantics=("parallel", "arbitrary"),
        ),
        interpret=interpret,
    )(x, gate_weight, bias2d, eg, eu, ed, sg, su, sd)
    return out


def kernel(hidden_states, gate_weight, e_score_correction_bias,
           expert_gate_w, expert_up_w, expert_down_w,
           shared_gate_w, shared_up_w, shared_down_w):
    orig_shape = hidden_states.shape
    x = hidden_states.reshape(-1, D).astype(jnp.float32)
    out = _moe(x, gate_weight, e_score_correction_bias.reshape(1, E),
               expert_gate_w, expert_up_w, expert_down_w,
               shared_gate_w, shared_up_w, shared_down_w)
    return out.reshape(orig_shape)


# speedup vs baseline: 1.5881x; 1.3666x over previous
import functools

import jax
import jax.numpy as jnp
from jax.experimental import pallas as pl
from jax.experimental.pallas import tpu as pltpu

E = 8
TOPK = 2
NG = 2
GS = E // NG
D = 1024
FF = 512
RSF = 2.5

TB = 256
NBLK = 2048 // TB


def _dot_t(a, b):
    return jax.lax.dot_general(a, b, (((1,), (1,)), ((), ())),
                               preferred_element_type=jnp.float32)


def _routing_combine(x, gw, bias):
    cols = jax.lax.broadcasted_iota(jnp.int32, (TB, E), 1)
    logits = _dot_t(x, gw)
    scores = jax.nn.sigmoid(logits)
    s_corr = scores + bias
    neg = jnp.float32(-jnp.inf)

    def top2sum(g):
        mask = (cols // GS) == g
        vals = jnp.where(mask, s_corr, neg)
        m1 = jnp.max(vals, axis=1, keepdims=True)
        eq = vals == m1
        neq = jnp.sum(eq.astype(jnp.float32), axis=1, keepdims=True)
        m2 = jnp.where(neq >= 2.0, m1,
                       jnp.max(jnp.where(eq, neg, vals), axis=1,
                               keepdims=True))
        return m1 + m2

    sel = jnp.where(top2sum(0) >= top2sum(1), 0, 1)
    v = jnp.where((cols // GS) == sel, s_corr, 0.0)

    big = jnp.int32(E + 1)
    m1v = jnp.max(v, axis=1, keepdims=True)
    idx1 = jnp.min(jnp.where(v == m1v, cols, big), axis=1, keepdims=True)
    is1 = cols == idx1
    w1 = jnp.sum(jnp.where(is1, scores, 0.0), axis=1, keepdims=True)
    v2 = jnp.where(is1, neg, v)
    m2v = jnp.max(v2, axis=1, keepdims=True)
    idx2 = jnp.min(jnp.where(v2 == m2v, cols, big), axis=1, keepdims=True)
    is2 = cols == idx2
    w2 = jnp.sum(jnp.where(is2, scores, 0.0), axis=1, keepdims=True)

    scale = jnp.float32(RSF) / (w1 + w2 + 1e-20)
    return (jnp.where(is1, w1, 0.0) + jnp.where(is2, w2, 0.0)) * scale


def _moe_body(x_ref, gw_ref, b_ref, eg_ref, eu_ref, ed_ref,
              sg_ref, su_ref, sd_ref, o_ref, acc_ref, comb_ref, xb_ref):
    e = pl.program_id(0)
    i = pl.program_id(1)
    rows = pl.ds(i * TB, TB)

    @pl.when(e == 0)
    def _init():
        x = x_ref[rows, :]
        comb_ref[rows, :] = _routing_combine(x, gw_ref[...], b_ref[...])
        xb = x.astype(jnp.bfloat16)
        xb_ref[rows, :] = xb
        hg = _dot_t(xb, sg_ref[...].astype(jnp.bfloat16))
        hu = _dot_t(xb, su_ref[...].astype(jnp.bfloat16))
        h = (jax.nn.silu(hg) * hu).astype(jnp.bfloat16)
        acc_ref[rows, :] = _dot_t(h, sd_ref[...].astype(jnp.bfloat16))

    xb = xb_ref[rows, :]
    hg = _dot_t(xb, eg_ref[0].astype(jnp.bfloat16))
    hu = _dot_t(xb, eu_ref[0].astype(jnp.bfloat16))
    h = (jax.nn.silu(hg) * hu).astype(jnp.bfloat16)
    eo = _dot_t(h, ed_ref[0].astype(jnp.bfloat16))
    cols = jax.lax.broadcasted_iota(jnp.int32, (TB, E), 1)
    ce = jnp.sum(jnp.where(cols == e, comb_ref[rows, :], 0.0), axis=1,
                 keepdims=True)
    acc_ref[rows, :] = acc_ref[rows, :] + eo * ce

    @pl.when(e == E - 1)
    def _fin():
        o_ref[...] = acc_ref[rows, :]


@jax.jit
def _moe(x, gate_weight, bias2d, eg, eu, ed, sg, su, sd):
    T = x.shape[0]
    grid = (E, NBLK)
    out = pl.pallas_call(
        _moe_body,
        grid=grid,
        in_specs=[
            pl.BlockSpec((T, D), lambda e, i: (0, 0)),
            pl.BlockSpec((E, D), lambda e, i: (0, 0)),
            pl.BlockSpec((1, E), lambda e, i: (0, 0)),
            pl.BlockSpec((1, FF, D), lambda e, i: (e, 0, 0)),
            pl.BlockSpec((1, FF, D), lambda e, i: (e, 0, 0)),
            pl.BlockSpec((1, D, FF), lambda e, i: (e, 0, 0)),
            pl.BlockSpec((FF, D), lambda e, i: (0, 0)),
            pl.BlockSpec((FF, D), lambda e, i: (0, 0)),
            pl.BlockSpec((D, FF), lambda e, i: (0, 0)),
        ],
        out_specs=pl.BlockSpec(
            (TB, D), lambda e, i: (jnp.where(e == E - 1, i, 0), 0)),
        out_shape=jax.ShapeDtypeStruct((T, D), jnp.float32),
        scratch_shapes=[
            pltpu.VMEM((T, D), jnp.float32),
            pltpu.VMEM((T, E), jnp.float32),
            pltpu.VMEM((T, D), jnp.bfloat16),
        ],
        compiler_params=pltpu.CompilerParams(
            dimension_semantics=("arbitrary", "arbitrary"),
        ),
    )(x, gate_weight, bias2d, eg, eu, ed, sg, su, sd)
    return out


def kernel(hidden_states, gate_weight, e_score_correction_bias,
           expert_gate_w, expert_up_w, expert_down_w,
           shared_gate_w, shared_up_w, shared_down_w):
    orig_shape = hidden_states.shape
    x = hidden_states.reshape(-1, D).astype(jnp.float32)
    out = _moe(x, gate_weight, e_score_correction_bias.reshape(1, E),
               expert_gate_w, expert_up_w, expert_down_w,
               shared_gate_w, shared_up_w, shared_down_w)
    return out.reshape(orig_shape)
